# baseline (speedup 1.0000x reference)
import functools

import jax
import jax.numpy as jnp
from jax import lax
from jax.experimental import pallas as pl
from jax.experimental.pallas import tpu as pltpu
from jax.experimental.pallas import tpu_sc as plsc

_CHUNK = 64
_L = 16
_NBUF = 12


@functools.lru_cache(maxsize=None)
def _make_kernel(NB: int, S: int, V: int, D: int):
    info = plsc.get_sparse_core_info()
    NC, NS = info.num_cores, info.num_subcores
    NW = NC * NS
    rows_per_w = (NB * S) // NW
    n_chunks = rows_per_w // _CHUNK
    wpb = S // rows_per_w
    mesh = plsc.VectorSubcoreMesh(core_axis_name="c", subcore_axis_name="s")

    @functools.partial(
        pl.kernel,
        out_type=jax.ShapeDtypeStruct((NB, S, D), jnp.float32),
        mesh=mesh,
        scratch_types=[
            pltpu.VMEM_SHARED((NS * V, D), jnp.float32),
            pltpu.VMEM((rows_per_w,), jnp.int32),
            pltpu.VMEM((_NBUF, _CHUNK, D), jnp.float32),
            pltpu.SemaphoreType.DMA,
            pltpu.SemaphoreType.DMA,
            pltpu.SemaphoreType.DMA,
            pltpu.SemaphoreType.DMA,
        ],
    )
    def k(ids_hbm, table_hbm, out_hbm, tab_sh, idx_v, bufs, sem_tab,
          sem_ids, gsem, wsem):
        cid = lax.axis_index("c")
        sid = lax.axis_index("s")
        wid = sid * NC + cid
        b = wid // wpb
        col = (wid % wpb) * rows_per_w
        ids_cp = pltpu.async_copy(
            ids_hbm.at[b, pl.ds(col, rows_per_w)], idx_v, sem_ids
        )
        off = sid * V
        tab_cp = pltpu.async_copy(
            table_hbm, tab_sh.at[pl.ds(off, V)], sem_tab
        )
        ids_cp.wait()
        for i in range(rows_per_w // _L):
            sl = pl.ds(i * _L, _L)
            idx_v[sl] = idx_v[sl] + off
        tab_cp.wait()

        def gather(c):
            return pltpu.async_copy(
                tab_sh.at[idx_v.at[pl.ds(c * _CHUNK, _CHUNK)]],
                bufs.at[c % _NBUF],
                gsem,
            )

        gcp = [None] * n_chunks
        wcp = [None] * n_chunks
        for c in range(min(_NBUF, n_chunks)):
            gcp[c] = gather(c)
        for c in range(n_chunks):
            if c >= _NBUF:
                wcp[c - _NBUF].wait()
                gcp[c] = gather(c)
            gcp[c].wait()
            wcp[c] = pltpu.async_copy(
                bufs.at[c % _NBUF],
                out_hbm.at[b, pl.ds(col + c * _CHUNK, _CHUNK)],
                wsem,
            )
        for c in range(max(0, n_chunks - _NBUF), n_chunks):
            wcp[c].wait()

    return k


def kernel(ids, table):
    NB, S = ids.shape
    V, D = table.shape
    k = _make_kernel(NB, S, V, D)
    return k(ids.astype(jnp.int32), table.astype(jnp.float32))


# speedup vs baseline: 1.0026x; 1.0026x over previous
import functools

import jax
import jax.numpy as jnp
from jax import lax
from jax.experimental import pallas as pl
from jax.experimental.pallas import tpu as pltpu
from jax.experimental.pallas import tpu_sc as plsc

_CHUNK = 128
_L = 16
_NBUF = 6


@functools.lru_cache(maxsize=None)
def _make_kernel(NB: int, S: int, V: int, D: int):
    info = plsc.get_sparse_core_info()
    NC, NS = info.num_cores, info.num_subcores
    NW = NC * NS
    rows_per_w = (NB * S) // NW
    n_chunks = rows_per_w // _CHUNK
    wpb = S // rows_per_w
    mesh = plsc.VectorSubcoreMesh(core_axis_name="c", subcore_axis_name="s")

    @functools.partial(
        pl.kernel,
        out_type=jax.ShapeDtypeStruct((NB, S, D), jnp.float32),
        mesh=mesh,
        scratch_types=[
            pltpu.VMEM_SHARED((NS * V, D), jnp.float32),
            pltpu.VMEM((rows_per_w,), jnp.int32),
            pltpu.VMEM((_NBUF, _CHUNK, D), jnp.float32),
            pltpu.SemaphoreType.DMA,
            pltpu.SemaphoreType.DMA,
            pltpu.SemaphoreType.DMA,
            pltpu.SemaphoreType.DMA,
        ],
    )
    def k(ids_hbm, table_hbm, out_hbm, tab_sh, idx_v, bufs, sem_tab,
          sem_ids, gsem, wsem):
        cid = lax.axis_index("c")
        sid = lax.axis_index("s")
        wid = sid * NC + cid
        b = wid // wpb
        col = (wid % wpb) * rows_per_w
        ids_cp = pltpu.async_copy(
            ids_hbm.at[b, pl.ds(col, rows_per_w)], idx_v, sem_ids
        )
        off = sid * V
        tab_cp = pltpu.async_copy(
            table_hbm, tab_sh.at[pl.ds(off, V)], sem_tab
        )
        ids_cp.wait()
        tab_cp.wait()
        tab_w = tab_sh.at[pl.ds(off, V)]

        def gather(c):
            return pltpu.async_copy(
                tab_w.at[idx_v.at[pl.ds(c * _CHUNK, _CHUNK)]],
                bufs.at[c % _NBUF],
                gsem,
            )

        gcp = [None] * n_chunks
        wcp = [None] * n_chunks
        for c in range(min(_NBUF, n_chunks)):
            gcp[c] = gather(c)
        for c in range(n_chunks):
            if c >= _NBUF:
                wcp[c - _NBUF].wait()
                gcp[c] = gather(c)
            gcp[c].wait()
            wcp[c] = pltpu.async_copy(
                bufs.at[c % _NBUF],
                out_hbm.at[b, pl.ds(col + c * _CHUNK, _CHUNK)],
                wsem,
            )
        for c in range(max(0, n_chunks - _NBUF), n_chunks):
            wcp[c].wait()

    return k


def kernel(ids, table):
    NB, S = ids.shape
    V, D = table.shape
    k = _make_kernel(NB, S, V, D)
    return k(ids.astype(jnp.int32), table.astype(jnp.float32))


# speedup vs baseline: 1.0048x; 1.0021x over previous
import functools

import jax
import jax.numpy as jnp
from jax import lax
from jax.experimental import pallas as pl
from jax.experimental.pallas import tpu as pltpu
from jax.experimental.pallas import tpu_sc as plsc

_CHUNK = 128
_L = 16
_NBUF = 6


@functools.lru_cache(maxsize=None)
def _make_kernel(NB: int, S: int, V: int, D: int):
    info = plsc.get_sparse_core_info()
    NC, NS = info.num_cores, info.num_subcores
    NW = NC * NS
    rows_per_w = (NB * S) // NW
    n_chunks = rows_per_w // _CHUNK
    wpb = S // rows_per_w
    mesh = plsc.VectorSubcoreMesh(core_axis_name="c", subcore_axis_name="s")

    @functools.partial(
        pl.kernel,
        out_type=jax.ShapeDtypeStruct((NB, S, D), jnp.float32),
        mesh=mesh,
        scratch_types=[
            pltpu.VMEM_SHARED((NS * V, D), jnp.float32),
            pltpu.VMEM((rows_per_w,), jnp.int32),
            pltpu.VMEM((_NBUF, _CHUNK, D), jnp.float32),
            pltpu.SemaphoreType.DMA,
            pltpu.SemaphoreType.DMA,
            pltpu.SemaphoreType.DMA,
            pltpu.SemaphoreType.DMA,
        ],
    )
    def k(ids_hbm, table_hbm, out_hbm, tab_sh, idx_v, bufs, sem_tab,
          sem_ids, gsem, wsem):
        cid = lax.axis_index("c")
        sid = lax.axis_index("s")
        wid = sid * NC + cid
        b = wid // wpb
        col = (wid % wpb) * rows_per_w
        ids_cp = pltpu.async_copy(
            ids_hbm.at[b, pl.ds(col, rows_per_w)], idx_v, sem_ids
        )
        off = sid * V
        tab_cp = pltpu.async_copy(
            table_hbm, tab_sh.at[pl.ds(off, V)], sem_tab
        )
        ids_cp.wait()
        for i in range(rows_per_w // _L):
            sl = pl.ds(i * _L, _L)
            idx_v[sl] = idx_v[sl] + off
        tab_cp.wait()

        def gather(c):
            return pltpu.async_copy(
                tab_sh.at[idx_v.at[pl.ds(c * _CHUNK, _CHUNK)]],
                bufs.at[c % _NBUF],
                gsem,
            )

        gcp = [None] * n_chunks
        wcp = [None] * n_chunks
        for c in range(min(_NBUF, n_chunks)):
            gcp[c] = gather(c)
        for c in range(n_chunks):
            if c >= _NBUF:
                wcp[c - _NBUF].wait()
                gcp[c] = gather(c)
            gcp[c].wait()
            wcp[c] = pltpu.async_copy(
                bufs.at[c % _NBUF],
                out_hbm.at[b, pl.ds(col + c * _CHUNK, _CHUNK)],
                wsem,
            )
        for c in range(max(0, n_chunks - _NBUF), n_chunks):
            wcp[c].wait()

    return k


def kernel(ids, table):
    NB, S = ids.shape
    V, D = table.shape
    k = _make_kernel(NB, S, V, D)
    return k(ids.astype(jnp.int32), table.astype(jnp.float32))


# speedup vs baseline: 1.0101x; 1.0054x over previous
import functools

import jax
import jax.numpy as jnp
from jax import lax
from jax.experimental import pallas as pl
from jax.experimental.pallas import tpu as pltpu
from jax.experimental.pallas import tpu_sc as plsc

_CHUNK = 128
_NBUF = 6


@functools.lru_cache(maxsize=None)
def _make_kernel(NB: int, S: int, V: int, D: int):
    info = plsc.get_sparse_core_info()
    NC, NS = info.num_cores, info.num_subcores
    NW = NC * NS
    rows_per_w = (NB * S) // NW
    n_chunks = rows_per_w // _CHUNK
    wpb = S // rows_per_w
    mesh = plsc.VectorSubcoreMesh(core_axis_name="c", subcore_axis_name="s")

    @functools.partial(
        pl.kernel,
        out_type=jax.ShapeDtypeStruct((NB, S, D), jnp.float32),
        mesh=mesh,
        scratch_types=[
            pltpu.VMEM_SHARED((NS * V, D), jnp.float32),
            pltpu.VMEM((rows_per_w,), jnp.int32),
            pltpu.VMEM((_NBUF, _CHUNK, D), jnp.float32),
            pltpu.SemaphoreType.DMA,
            pltpu.SemaphoreType.DMA,
            [pltpu.SemaphoreType.DMA] * _NBUF,
            [pltpu.SemaphoreType.DMA] * _NBUF,
        ],
    )
    def k(ids_hbm, table_hbm, out_hbm, tab_sh, idx_v, bufs, sem_tab,
          sem_ids, gsems, wsems):
        cid = lax.axis_index("c")
        sid = lax.axis_index("s")
        wid = sid * NC + cid
        b = wid // wpb
        col = (wid % wpb) * rows_per_w
        ids_cp = pltpu.async_copy(
            ids_hbm.at[b, pl.ds(col, rows_per_w)], idx_v, sem_ids
        )
        off = sid * V
        tab_cp = pltpu.async_copy(
            table_hbm, tab_sh.at[pl.ds(off, V)], sem_tab
        )
        ids_cp.wait()
        tab_cp.wait()
        tab_w = tab_sh.at[pl.ds(off, V)]

        def gather(c):
            return pltpu.async_copy(
                tab_w.at[idx_v.at[pl.ds(c * _CHUNK, _CHUNK)]],
                bufs.at[c % _NBUF],
                gsems[c % _NBUF],
            )

        gcp = [None] * n_chunks
        wcp = [None] * n_chunks
        for c in range(min(_NBUF, n_chunks)):
            gcp[c] = gather(c)
        for c in range(n_chunks):
            if c >= _NBUF:
                wcp[c - _NBUF].wait()
                gcp[c] = gather(c)
            gcp[c].wait()
            wcp[c] = pltpu.async_copy(
                bufs.at[c % _NBUF],
                out_hbm.at[b, pl.ds(col + c * _CHUNK, _CHUNK)],
                wsems[c % _NBUF],
            )
        for c in range(max(0, n_chunks - _NBUF), n_chunks):
            wcp[c].wait()

    return k


def kernel(ids, table):
    NB, S = ids.shape
    V, D = table.shape
    k = _make_kernel(NB, S, V, D)
    return k(ids.astype(jnp.int32), table.astype(jnp.float32))
